# baseline (speedup 1.0000x reference)
import functools

import jax
import jax.numpy as jnp
from jax import lax
from jax.experimental import pallas as pl
from jax.experimental.pallas import tpu as pltpu
from jax.experimental.pallas import tpu_sc as plsc

N = 10000
E = 320000
D = 128
HD = D // 2

NC = 2
NS = 16
NW = NC * NS
C = 80
DCH = E // NW // C
HC = 128
HCH = 160
ESUB = E // NS
EPAD = HCH * HC - ESUB
NPAD = 10240
RPS = NPAD // NS
DC = 128
NDC = RPS // DC

_MESH = dict(core_axis_name="c", subcore_axis_name="s", num_cores=NC,
             num_subcores=NS)


@functools.partial(
    pl.kernel,
    out_type=jax.ShapeDtypeStruct((NC * NPAD,), jnp.float32),
    mesh=plsc.VectorSubcoreMesh(**_MESH),
    scratch_types=[
        pltpu.VMEM((DCH, C), jnp.int32),
        pltpu.VMEM((C,), jnp.float32),
        pltpu.VMEM((RPS,), jnp.float32),
        pltpu.VMEM_SHARED((NPAD,), jnp.float32),
    ],
)
def _deg_kernel(dst_hbm, out_hbm, dst_v, ones_v, stage_v, acc):
    c = lax.axis_index("c")
    s = lax.axis_index("s")
    wid = s * NC + c

    one16 = jnp.full((16,), 1.0, jnp.float32)
    zero16 = jnp.zeros((16,), jnp.float32)
    for t in range(C // 16):
        ones_v[pl.ds(t * 16, 16)] = one16

    def zfill(i, carry):
        stage_v[pl.ds(i * 16, 16)] = zero16
        return carry

    lax.fori_loop(0, RPS // 16, zfill, 0)
    pltpu.sync_copy(stage_v, acc.at[pl.ds(s * RPS, RPS)])
    pltpu.sync_copy(dst_hbm.at[wid], dst_v)
    plsc.subcore_barrier()

    def body(j, carry):
        pltpu.sync_copy(ones_v, acc.at[dst_v.at[j]], add=True)
        return carry

    lax.fori_loop(0, DCH, body, 0)
    plsc.subcore_barrier()

    pltpu.sync_copy(acc.at[pl.ds(s * RPS, RPS)], stage_v)
    pltpu.sync_copy(stage_v, out_hbm.at[pl.ds(c * NPAD + s * RPS, RPS)])


@functools.partial(
    pl.kernel,
    out_type=jax.ShapeDtypeStruct((NC * NPAD, HD), jnp.float32),
    mesh=plsc.VectorSubcoreMesh(**_MESH),
    scratch_types=[
        pltpu.VMEM((HCH, HC), jnp.int32),
        pltpu.VMEM((HCH, HC), jnp.int32),
        pltpu.VMEM((HC, HD), jnp.float32),
        pltpu.VMEM((HC, HD), jnp.float32),
        pltpu.VMEM((DC, HD), jnp.float32),
        pltpu.VMEM_SHARED((NPAD, HD), jnp.float32),
        pltpu.SemaphoreType.DMA,
        pltpu.SemaphoreType.DMA,
    ],
    compiler_params=pltpu.CompilerParams(use_tc_tiling_on_sc=False),
)
def _hop_kernel(g_hbm, src_hbm, dst_hbm, out_hbm, src_v, dst_v, rows0,
                rows1, stage_v, acc, sem0, sem1):
    c = lax.axis_index("c")
    s = lax.axis_index("s")

    zero16 = jnp.zeros((16,), jnp.float32)

    def zrow(r, carry):
        for t in range(HD // 16):
            stage_v[r, pl.ds(t * 16, 16)] = zero16
        return carry

    lax.fori_loop(0, DC, zrow, 0)
    for k in range(NDC):
        pltpu.sync_copy(stage_v, acc.at[pl.ds(s * RPS + k * DC, DC)])
    pltpu.sync_copy(src_hbm.at[s], src_v)
    pltpu.sync_copy(dst_hbm.at[s], dst_v)

    off16 = jnp.zeros((16,), jnp.int32) + c * N

    def shift(r, carry):
        for t in range(HC // 16):
            src_v[r, pl.ds(t * 16, 16)] = src_v[r, pl.ds(t * 16, 16)] + off16
        return carry

    lax.fori_loop(0, HCH, shift, 0)
    plsc.subcore_barrier()

    pltpu.async_copy(g_hbm.at[src_v.at[0]], rows0, sem0)

    def body(i, carry):
        a = 2 * i
        pltpu.async_copy(g_hbm.at[src_v.at[a + 1]], rows1, sem1)
        pltpu.make_async_copy(g_hbm.at[src_v.at[a]], rows0, sem0).wait()
        pltpu.sync_copy(rows0, acc.at[dst_v.at[a]], add=True)
        nxt = jnp.minimum(a + 2, HCH - 1)
        pltpu.async_copy(g_hbm.at[src_v.at[nxt]], rows0, sem0)
        pltpu.make_async_copy(g_hbm.at[src_v.at[a + 1]], rows1, sem1).wait()
        pltpu.sync_copy(rows1, acc.at[dst_v.at[a + 1]], add=True)
        return carry

    lax.fori_loop(0, HCH // 2, body, 0)
    pltpu.make_async_copy(g_hbm.at[src_v.at[HCH - 1]], rows0, sem0).wait()
    plsc.subcore_barrier()

    for k in range(NDC):
        pltpu.sync_copy(acc.at[pl.ds(s * RPS + k * DC, DC)], stage_v)
        pltpu.sync_copy(
            stage_v, out_hbm.at[pl.ds(c * NPAD + s * RPS + k * DC, DC)])


_R = 1000


def _mm_body(x_ref, w_ref, d_ref, o_ref):
    deg = jnp.maximum(d_ref[0] + d_ref[1], 1.0)
    res = jnp.dot(x_ref[...], w_ref[...],
                  preferred_element_type=jnp.float32) * lax.rsqrt(deg)
    o_ref[0] = res[:, :HD]
    o_ref[1] = res[:, HD:]


def _comb_body(p_ref, d_ref, o_ref):
    deg = jnp.maximum(d_ref[0] + d_ref[1], 1.0)
    o_ref[...] = p_ref[...] / deg


def _fin_body(q_ref, d_ref, b_ref, o_ref):
    deg = jnp.maximum(d_ref[0] + d_ref[1], 1.0)
    norm = lax.rsqrt(deg)
    o_ref[...] = (jnp.concatenate([q_ref[0], q_ref[1]], axis=1) * norm
                  + b_ref[...])


def _scaled_mm(feat, W, d3):
    return pl.pallas_call(
        _mm_body,
        grid=(N // _R,),
        in_specs=[
            pl.BlockSpec((_R, D), lambda i: (i, 0)),
            pl.BlockSpec((D, D), lambda i: (0, 0)),
            pl.BlockSpec((NC, _R, 1), lambda i: (0, i, 0)),
        ],
        out_specs=pl.BlockSpec((NC, _R, HD), lambda i: (0, i, 0)),
        out_shape=jax.ShapeDtypeStruct((NC, N, HD), jnp.float32),
    )(feat, W, d3)


def _combine(p, d3):
    return pl.pallas_call(
        _comb_body,
        grid=(N // _R,),
        in_specs=[
            pl.BlockSpec((NC, _R, HD), lambda i: (0, i, 0)),
            pl.BlockSpec((NC, _R, 1), lambda i: (0, i, 0)),
        ],
        out_specs=pl.BlockSpec((NC, _R, HD), lambda i: (0, i, 0)),
        out_shape=jax.ShapeDtypeStruct((NC, N, HD), jnp.float32),
    )(p, d3)


def _finalize(q, d3, b2):
    return pl.pallas_call(
        _fin_body,
        grid=(N // _R,),
        in_specs=[
            pl.BlockSpec((NC, _R, HD), lambda i: (0, i, 0)),
            pl.BlockSpec((NC, _R, 1), lambda i: (0, i, 0)),
            pl.BlockSpec((1, D), lambda i: (0, 0)),
        ],
        out_specs=pl.BlockSpec((_R, D), lambda i: (i, 0)),
        out_shape=jax.ShapeDtypeStruct((N, D), jnp.float32),
    )(q, d3, b2)


def kernel(feat, edge_index, W, b):
    src = jnp.pad(edge_index[0].reshape(NS, ESUB),
                  ((0, 0), (0, EPAD))).reshape(NS, HCH, HC)
    trash = jnp.broadcast_to(N + (jnp.arange(EPAD) % (NPAD - N)), (NS, EPAD))
    dst_h = jnp.concatenate([edge_index[1].reshape(NS, ESUB), trash],
                            axis=1).reshape(NS, HCH, HC)
    dst_d = edge_index[1].reshape(NW, DCH, C)
    dpart = _deg_kernel(dst_d)
    d3 = dpart.reshape(NC, NPAD, 1)
    g0 = _scaled_mm(feat, W, d3)
    p = _hop_kernel(g0.reshape(NC * N, HD), src, dst_h)
    g1 = _combine(p.reshape(NC, NPAD, HD), d3)
    q = _hop_kernel(g1.reshape(NC * N, HD), src, dst_h)
    return _finalize(q.reshape(NC, NPAD, HD), d3, b.reshape(1, D))


# speedup vs baseline: 1.8496x; 1.8496x over previous
import functools

import jax
import jax.numpy as jnp
from jax import lax
from jax.experimental import pallas as pl
from jax.experimental.pallas import tpu as pltpu
from jax.experimental.pallas import tpu_sc as plsc

N = 10000
E = 320000
D = 128
HD = D // 2

NC = 2
NS = 16
NW = NC * NS
C = 80
DCH = E // NW // C
HC = 80
HCH = 250
ESUB = E // NS
EPAD = HCH * HC - ESUB
NPAD = 10240
RPS = NPAD // NS
DC = 128
NDC = RPS // DC

_MESH = dict(core_axis_name="c", subcore_axis_name="s", num_cores=NC,
             num_subcores=NS)


@functools.partial(
    pl.kernel,
    out_type=jax.ShapeDtypeStruct((NC * NPAD,), jnp.float32),
    mesh=plsc.VectorSubcoreMesh(**_MESH),
    scratch_types=[
        pltpu.VMEM((DCH, C), jnp.int32),
        pltpu.VMEM((C,), jnp.float32),
        pltpu.VMEM((RPS,), jnp.float32),
        pltpu.VMEM_SHARED((NPAD,), jnp.float32),
    ],
)
def _deg_kernel(dst_hbm, out_hbm, dst_v, ones_v, stage_v, acc):
    c = lax.axis_index("c")
    s = lax.axis_index("s")
    wid = s * NC + c

    one16 = jnp.full((16,), 1.0, jnp.float32)
    zero16 = jnp.zeros((16,), jnp.float32)
    for t in range(C // 16):
        ones_v[pl.ds(t * 16, 16)] = one16

    def zfill(i, carry):
        stage_v[pl.ds(i * 16, 16)] = zero16
        return carry

    lax.fori_loop(0, RPS // 16, zfill, 0)
    pltpu.sync_copy(stage_v, acc.at[pl.ds(s * RPS, RPS)])
    pltpu.sync_copy(dst_hbm.at[wid], dst_v)
    plsc.subcore_barrier()

    def body(j, carry):
        pltpu.sync_copy(ones_v, acc.at[dst_v.at[j]], add=True)
        return carry

    lax.fori_loop(0, DCH, body, 0)
    plsc.subcore_barrier()

    pltpu.sync_copy(acc.at[pl.ds(s * RPS, RPS)], stage_v)
    pltpu.sync_copy(stage_v, out_hbm.at[pl.ds(c * NPAD + s * RPS, RPS)])


@functools.partial(
    pl.kernel,
    out_type=jax.ShapeDtypeStruct((NC * NPAD, HD), jnp.float32),
    mesh=plsc.VectorSubcoreMesh(**_MESH),
    scratch_types=[
        pltpu.VMEM((HCH, HC), jnp.int32),
        pltpu.VMEM((HCH, HC), jnp.int32),
        [pltpu.VMEM((HC, HD), jnp.float32) for _ in range(4)],
        pltpu.VMEM((DC, HD), jnp.float32),
        pltpu.VMEM_SHARED((NPAD, HD), jnp.float32),
        [pltpu.SemaphoreType.DMA for _ in range(4)],
        [pltpu.SemaphoreType.DMA for _ in range(4)],
    ],
    compiler_params=pltpu.CompilerParams(use_tc_tiling_on_sc=False),
)
def _hop_kernel(g_hbm, src_hbm, dst_hbm, out_hbm, src_v, dst_v, rows,
                stage_v, acc, gs, ss):
    c = lax.axis_index("c")
    s = lax.axis_index("s")

    zero16 = jnp.zeros((16,), jnp.float32)

    def zrow(r, carry):
        for t in range(HD // 16):
            stage_v[r, pl.ds(t * 16, 16)] = zero16
        return carry

    lax.fori_loop(0, DC, zrow, 0)
    for k in range(NDC):
        pltpu.sync_copy(stage_v, acc.at[pl.ds(s * RPS + k * DC, DC)])
    pltpu.sync_copy(src_hbm.at[s], src_v)
    pltpu.sync_copy(dst_hbm.at[s], dst_v)

    off16 = jnp.zeros((16,), jnp.int32) + c * N

    def shift(r, carry):
        for t in range(HC // 16):
            src_v[r, pl.ds(t * 16, 16)] = src_v[r, pl.ds(t * 16, 16)] + off16
        return carry

    lax.fori_loop(0, HCH, shift, 0)
    plsc.subcore_barrier()

    def wait_g(k):
        pltpu.make_async_copy(g_hbm.at[src_v.at[0]], rows[k], gs[k]).wait()

    def wait_s(k):
        pltpu.make_async_copy(rows[k], acc.at[dst_v.at[0]], ss[k]).wait()

    def gather(j, k):
        pltpu.async_copy(g_hbm.at[src_v.at[j]], rows[k], gs[k])

    def scatter(j, k):
        pltpu.async_copy(rows[k], acc.at[dst_v.at[j]], ss[k], add=True)

    gather(0, 0)
    gather(1, 1)
    for j in (0, 1):
        wait_g(j)
        scatter(j, j)
        gather(j + 2, j + 2)

    def body(i, carry):
        for m in range(4):
            j = 2 + 4 * i + m
            k = (2 + m) % 4
            k2 = (k + 2) % 4
            wait_g(k)
            scatter(j, k)
            wait_s(k2)
            gather(jnp.minimum(j + 2, HCH - 1), k2)
        return carry

    lax.fori_loop(0, (HCH - 2) // 4, body, 0)
    for k in (0, 1):
        wait_s(k)
    for k in (2, 3):
        wait_g(k)
    plsc.subcore_barrier()

    for k in range(NDC):
        pltpu.sync_copy(acc.at[pl.ds(s * RPS + k * DC, DC)], stage_v)
        pltpu.sync_copy(
            stage_v, out_hbm.at[pl.ds(c * NPAD + s * RPS + k * DC, DC)])


_R = 1000


def _mm_body(x_ref, w_ref, d_ref, o_ref):
    deg = jnp.maximum(d_ref[0] + d_ref[1], 1.0)
    res = jnp.dot(x_ref[...], w_ref[...],
                  preferred_element_type=jnp.float32) * lax.rsqrt(deg)
    o_ref[0] = res[:, :HD]
    o_ref[1] = res[:, HD:]


def _comb_body(p_ref, d_ref, o_ref):
    deg = jnp.maximum(d_ref[0] + d_ref[1], 1.0)
    o_ref[...] = p_ref[...] / deg


def _fin_body(q_ref, d_ref, b_ref, o_ref):
    deg = jnp.maximum(d_ref[0] + d_ref[1], 1.0)
    norm = lax.rsqrt(deg)
    o_ref[...] = (jnp.concatenate([q_ref[0], q_ref[1]], axis=1) * norm
                  + b_ref[...])


def _scaled_mm(feat, W, d3):
    return pl.pallas_call(
        _mm_body,
        grid=(N // _R,),
        in_specs=[
            pl.BlockSpec((_R, D), lambda i: (i, 0)),
            pl.BlockSpec((D, D), lambda i: (0, 0)),
            pl.BlockSpec((NC, _R, 1), lambda i: (0, i, 0)),
        ],
        out_specs=pl.BlockSpec((NC, _R, HD), lambda i: (0, i, 0)),
        out_shape=jax.ShapeDtypeStruct((NC, N, HD), jnp.float32),
    )(feat, W, d3)


def _combine(p, d3):
    return pl.pallas_call(
        _comb_body,
        grid=(N // _R,),
        in_specs=[
            pl.BlockSpec((NC, _R, HD), lambda i: (0, i, 0)),
            pl.BlockSpec((NC, _R, 1), lambda i: (0, i, 0)),
        ],
        out_specs=pl.BlockSpec((NC, _R, HD), lambda i: (0, i, 0)),
        out_shape=jax.ShapeDtypeStruct((NC, N, HD), jnp.float32),
    )(p, d3)


def _finalize(q, d3, b2):
    return pl.pallas_call(
        _fin_body,
        grid=(N // _R,),
        in_specs=[
            pl.BlockSpec((NC, _R, HD), lambda i: (0, i, 0)),
            pl.BlockSpec((NC, _R, 1), lambda i: (0, i, 0)),
            pl.BlockSpec((1, D), lambda i: (0, 0)),
        ],
        out_specs=pl.BlockSpec((_R, D), lambda i: (i, 0)),
        out_shape=jax.ShapeDtypeStruct((N, D), jnp.float32),
    )(q, d3, b2)


def kernel(feat, edge_index, W, b):
    src = edge_index[0].reshape(NS, HCH, HC)
    dst_h = edge_index[1].reshape(NS, HCH, HC)
    dst_d = edge_index[1].reshape(NW, DCH, C)
    dpart = _deg_kernel(dst_d)
    d3 = dpart.reshape(NC, NPAD, 1)
    g0 = _scaled_mm(feat, W, d3)
    p = _hop_kernel(g0.reshape(NC * N, HD), src, dst_h)
    g1 = _combine(p.reshape(NC, NPAD, HD), d3)
    q = _hop_kernel(g1.reshape(NC * N, HD), src, dst_h)
    return _finalize(q.reshape(NC, NPAD, HD), d3, b.reshape(1, D))


# speedup vs baseline: 1.9042x; 1.0296x over previous
import functools

import jax
import jax.numpy as jnp
from jax import lax
from jax.experimental import pallas as pl
from jax.experimental.pallas import tpu as pltpu
from jax.experimental.pallas import tpu_sc as plsc

N = 10000
E = 320000
D = 128
HD = D // 2

NC = 2
NS = 16
NW = NC * NS
C = 80
DCH = E // NW // C
HC = 80
HCH = E // NS // HC
NPAD = 10240
RPS = NPAD // NS
RPS2 = N // NS
DRC = 125
NDR = RPS2 // DRC

_MESH = dict(core_axis_name="c", subcore_axis_name="s", num_cores=NC,
             num_subcores=NS)


@functools.partial(
    pl.kernel,
    out_type=jax.ShapeDtypeStruct((NC * NPAD,), jnp.float32),
    mesh=plsc.VectorSubcoreMesh(**_MESH),
    scratch_types=[
        pltpu.VMEM((DCH, C), jnp.int32),
        pltpu.VMEM((C,), jnp.float32),
        pltpu.VMEM((RPS,), jnp.float32),
        pltpu.VMEM_SHARED((NPAD,), jnp.float32),
    ],
)
def _deg_kernel(dst_hbm, out_hbm, dst_v, ones_v, stage_v, acc):
    c = lax.axis_index("c")
    s = lax.axis_index("s")
    wid = s * NC + c

    one16 = jnp.full((16,), 1.0, jnp.float32)
    zero16 = jnp.zeros((16,), jnp.float32)
    for t in range(C // 16):
        ones_v[pl.ds(t * 16, 16)] = one16

    def zfill(i, carry):
        stage_v[pl.ds(i * 16, 16)] = zero16
        return carry

    lax.fori_loop(0, RPS // 16, zfill, 0)
    pltpu.sync_copy(stage_v, acc.at[pl.ds(s * RPS, RPS)])
    pltpu.sync_copy(dst_hbm.at[wid], dst_v)
    plsc.subcore_barrier()

    def body(j, carry):
        pltpu.sync_copy(ones_v, acc.at[dst_v.at[j]], add=True)
        return carry

    lax.fori_loop(0, DCH, body, 0)
    plsc.subcore_barrier()

    pltpu.sync_copy(acc.at[pl.ds(s * RPS, RPS)], stage_v)
    pltpu.sync_copy(stage_v, out_hbm.at[pl.ds(c * NPAD + s * RPS, RPS)])


def _hop_body(final, g_hbm, src_hbm, dst_hbm, scale_hbm, bias_hbm, out_hbm,
              src_v, dst_v, scale_v, bias_v, rows, stage_v, acc, gs, ss):
    c = lax.axis_index("c")
    s = lax.axis_index("s")

    zero16 = jnp.zeros((16,), jnp.float32)

    def zrow(r, carry):
        for t in range(HD // 16):
            stage_v[r, pl.ds(t * 16, 16)] = zero16
        return carry

    lax.fori_loop(0, DRC, zrow, 0)
    for k in range(NDR):
        pltpu.sync_copy(stage_v, acc.at[pl.ds(s * RPS2 + k * DRC, DRC)])
    pltpu.sync_copy(src_hbm.at[s], src_v)
    pltpu.sync_copy(dst_hbm.at[s], dst_v)
    pltpu.sync_copy(bias_hbm.at[c], bias_v)

    off16 = jnp.zeros((16,), jnp.int32) + c * N

    def shift(r, carry):
        for t in range(HC // 16):
            src_v[r, pl.ds(t * 16, 16)] = src_v[r, pl.ds(t * 16, 16)] + off16
        return carry

    lax.fori_loop(0, HCH, shift, 0)
    plsc.subcore_barrier()

    def wait_g(k):
        pltpu.make_async_copy(g_hbm.at[src_v.at[0]], rows[k], gs[k]).wait()

    def wait_s(k):
        pltpu.make_async_copy(rows[k], acc.at[dst_v.at[0]], ss[k]).wait()

    def gather(j, k):
        pltpu.async_copy(g_hbm.at[src_v.at[j]], rows[k], gs[k])

    def scatter(j, k):
        pltpu.async_copy(rows[k], acc.at[dst_v.at[j]], ss[k], add=True)

    gather(0, 0)
    gather(1, 1)
    for j in (0, 1):
        wait_g(j)
        scatter(j, j)
        gather(j + 2, j + 2)

    def body(i, carry):
        for m in range(4):
            j = 2 + 4 * i + m
            k = (2 + m) % 4
            k2 = (k + 2) % 4
            wait_g(k)
            scatter(j, k)
            wait_s(k2)
            gather(jnp.minimum(j + 2, HCH - 1), k2)
        return carry

    lax.fori_loop(0, (HCH - 2) // 4, body, 0)
    for k in (0, 1):
        wait_s(k)
    for k in (2, 3):
        wait_g(k)
    plsc.subcore_barrier()

    for k in range(NDR):
        base = s * RPS2 + k * DRC
        pltpu.sync_copy(acc.at[pl.ds(base, DRC)], stage_v)
        pltpu.sync_copy(scale_hbm.at[pl.ds(base, DRC)], scale_v)

        def scrow(r, carry):
            svec = scale_v[r]
            for t in range(HD // 16):
                sl = pl.ds(t * 16, 16)
                stage_v[r, sl] = stage_v[r, sl] * svec + bias_v[sl]
            return carry

        lax.fori_loop(0, DRC, scrow, 0)
        if final:
            pltpu.sync_copy(
                stage_v, out_hbm.at[pl.ds(base, DRC), pl.ds(c * HD, HD)])
        else:
            pltpu.sync_copy(stage_v, out_hbm.at[pl.ds(c * N + base, DRC)])


def _make_hop(final):
    return functools.partial(
        pl.kernel,
        out_type=jax.ShapeDtypeStruct(
            (N, D) if final else (NC * N, HD), jnp.float32),
        mesh=plsc.VectorSubcoreMesh(**_MESH),
        scratch_types=[
            pltpu.VMEM((HCH, HC), jnp.int32),
            pltpu.VMEM((HCH, HC), jnp.int32),
            pltpu.VMEM((DRC, 16), jnp.float32),
            pltpu.VMEM((HD,), jnp.float32),
            [pltpu.VMEM((HC, HD), jnp.float32) for _ in range(4)],
            pltpu.VMEM((DRC, HD), jnp.float32),
            pltpu.VMEM_SHARED((N, HD), jnp.float32),
            [pltpu.SemaphoreType.DMA for _ in range(4)],
            [pltpu.SemaphoreType.DMA for _ in range(4)],
        ],
        compiler_params=pltpu.CompilerParams(use_tc_tiling_on_sc=False),
    )(functools.partial(_hop_body, final))


_hop_mid = _make_hop(False)
_hop_fin = _make_hop(True)

_R = 1000


def _mm_body(x_ref, w_ref, d_ref, o_ref, inv_ref, nrm_ref):
    deg = jnp.maximum(d_ref[0] + d_ref[1], 1.0)
    nrm = lax.rsqrt(deg)
    res = jnp.dot(x_ref[...], w_ref[...],
                  preferred_element_type=jnp.float32) * nrm
    o_ref[0] = res[:, :HD]
    o_ref[1] = res[:, HD:]
    inv_ref[...] = jnp.broadcast_to(1.0 / deg, (_R, 16))
    nrm_ref[...] = jnp.broadcast_to(nrm, (_R, 16))


def _scaled_mm(feat, W, d3):
    return pl.pallas_call(
        _mm_body,
        grid=(N // _R,),
        in_specs=[
            pl.BlockSpec((_R, D), lambda i: (i, 0)),
            pl.BlockSpec((D, D), lambda i: (0, 0)),
            pl.BlockSpec((NC, _R, 1), lambda i: (0, i, 0)),
        ],
        out_specs=[
            pl.BlockSpec((NC, _R, HD), lambda i: (0, i, 0)),
            pl.BlockSpec((_R, 16), lambda i: (i, 0)),
            pl.BlockSpec((_R, 16), lambda i: (i, 0)),
        ],
        out_shape=[
            jax.ShapeDtypeStruct((NC, N, HD), jnp.float32),
            jax.ShapeDtypeStruct((N, 16), jnp.float32),
            jax.ShapeDtypeStruct((N, 16), jnp.float32),
        ],
    )(feat, W, d3)


def kernel(feat, edge_index, W, b):
    src = edge_index[0].reshape(NS, HCH, HC)
    dst_h = edge_index[1].reshape(NS, HCH, HC)
    dst_d = edge_index[1].reshape(NW, DCH, C)
    dpart = _deg_kernel(dst_d)
    d3 = dpart.reshape(NC, NPAD, 1)
    g0, inv, nrm = _scaled_mm(feat, W, d3)
    zeros2 = jnp.zeros((NC, HD), jnp.float32)
    p = _hop_mid(g0.reshape(NC * N, HD), src, dst_h, inv, zeros2)
    return _hop_fin(p, src, dst_h, nrm, b.reshape(NC, HD))


# speedup vs baseline: 1.9104x; 1.0033x over previous
import functools

import jax
import jax.numpy as jnp
from jax import lax
from jax.experimental import pallas as pl
from jax.experimental.pallas import tpu as pltpu
from jax.experimental.pallas import tpu_sc as plsc

N = 10000
E = 320000
D = 128
HD = D // 2

NC = 2
NS = 16
NW = NC * NS
C = 80
DCH = E // NW // C
HC = 80
HCH = E // NS // HC
NPAD = 10240
RPS = NPAD // NS
RPS2 = N // NS
DRC = 125
NDR = RPS2 // DRC

_MESH = dict(core_axis_name="c", subcore_axis_name="s", num_cores=NC,
             num_subcores=NS)


@functools.partial(
    pl.kernel,
    out_type=jax.ShapeDtypeStruct((NC * NPAD,), jnp.float32),
    mesh=plsc.VectorSubcoreMesh(**_MESH),
    scratch_types=[
        pltpu.VMEM((DCH, C), jnp.int32),
        pltpu.VMEM((C,), jnp.float32),
        pltpu.VMEM((RPS,), jnp.float32),
        pltpu.VMEM_SHARED((NPAD,), jnp.float32),
    ],
)
def _deg_kernel(dst_hbm, out_hbm, dst_v, ones_v, stage_v, acc):
    c = lax.axis_index("c")
    s = lax.axis_index("s")
    wid = s * NC + c

    one16 = jnp.full((16,), 1.0, jnp.float32)
    zero16 = jnp.zeros((16,), jnp.float32)
    for t in range(C // 16):
        ones_v[pl.ds(t * 16, 16)] = one16

    def zfill(i, carry):
        stage_v[pl.ds(i * 16, 16)] = zero16
        return carry

    lax.fori_loop(0, RPS // 16, zfill, 0)
    pltpu.sync_copy(stage_v, acc.at[pl.ds(s * RPS, RPS)])
    pltpu.sync_copy(dst_hbm.at[wid], dst_v)
    plsc.subcore_barrier()

    def body(j, carry):
        pltpu.sync_copy(ones_v, acc.at[dst_v.at[j]], add=True)
        return carry

    lax.fori_loop(0, DCH, body, 0)
    plsc.subcore_barrier()

    pltpu.sync_copy(acc.at[pl.ds(s * RPS, RPS)], stage_v)
    pltpu.sync_copy(stage_v, out_hbm.at[pl.ds(c * NPAD + s * RPS, RPS)])


def _hop_body(final, g_hbm, src_hbm, dst_hbm, scale_hbm, bias_hbm, out_hbm,
              src_v, dst_v, scale_v, bias_v, rows, stage_v, acc, gs, ss):
    c = lax.axis_index("c")
    s = lax.axis_index("s")

    zero16 = jnp.zeros((16,), jnp.float32)

    def zrow(r, carry):
        for t in range(HD // 16):
            stage_v[r, pl.ds(t * 16, 16)] = zero16
        return carry

    lax.fori_loop(0, DRC, zrow, 0)
    for k in range(NDR):
        pltpu.sync_copy(stage_v, acc.at[pl.ds(s * RPS2 + k * DRC, DRC)])
    pltpu.sync_copy(src_hbm.at[c * NS + s], src_v)
    pltpu.sync_copy(dst_hbm.at[s], dst_v)
    pltpu.sync_copy(bias_hbm.at[c], bias_v)
    plsc.subcore_barrier()

    def wait_g(k):
        pltpu.make_async_copy(g_hbm.at[src_v.at[0]], rows[k], gs[k]).wait()

    def wait_s(k):
        pltpu.make_async_copy(rows[k], acc.at[dst_v.at[0]], ss[k]).wait()

    def gather(j, k):
        pltpu.async_copy(g_hbm.at[src_v.at[j]], rows[k], gs[k])

    def scatter(j, k):
        pltpu.async_copy(rows[k], acc.at[dst_v.at[j]], ss[k], add=True)

    gather(0, 0)
    gather(1, 1)
    for j in (0, 1):
        wait_g(j)
        scatter(j, j)
        gather(j + 2, j + 2)

    def body(i, carry):
        for m in range(4):
            j = 2 + 4 * i + m
            k = (2 + m) % 4
            k2 = (k + 2) % 4
            wait_g(k)
            scatter(j, k)
            wait_s(k2)
            gather(jnp.minimum(j + 2, HCH - 1), k2)
        return carry

    lax.fori_loop(0, (HCH - 2) // 4, body, 0)
    for k in (0, 1):
        wait_s(k)
    for k in (2, 3):
        wait_g(k)
    plsc.subcore_barrier()

    for k in range(NDR):
        base = s * RPS2 + k * DRC
        pltpu.sync_copy(acc.at[pl.ds(base, DRC)], stage_v)
        pltpu.sync_copy(scale_hbm.at[pl.ds(base, DRC)], scale_v)

        def scrow(r, carry):
            svec = scale_v[r]
            for t in range(HD // 16):
                sl = pl.ds(t * 16, 16)
                stage_v[r, sl] = stage_v[r, sl] * svec + bias_v[sl]
            return carry

        lax.fori_loop(0, DRC, scrow, 0)
        if final:
            pltpu.sync_copy(
                stage_v, out_hbm.at[pl.ds(base, DRC), pl.ds(c * HD, HD)])
        else:
            pltpu.sync_copy(stage_v, out_hbm.at[pl.ds(c * N + base, DRC)])


def _make_hop(final):
    return functools.partial(
        pl.kernel,
        out_type=jax.ShapeDtypeStruct(
            (N, D) if final else (NC * N, HD), jnp.float32),
        mesh=plsc.VectorSubcoreMesh(**_MESH),
        scratch_types=[
            pltpu.VMEM((HCH, HC), jnp.int32),
            pltpu.VMEM((HCH, HC), jnp.int32),
            pltpu.VMEM((DRC, 16), jnp.float32),
            pltpu.VMEM((HD,), jnp.float32),
            [pltpu.VMEM((HC, HD), jnp.float32) for _ in range(4)],
            pltpu.VMEM((DRC, HD), jnp.float32),
            pltpu.VMEM_SHARED((N, HD), jnp.float32),
            [pltpu.SemaphoreType.DMA for _ in range(4)],
            [pltpu.SemaphoreType.DMA for _ in range(4)],
        ],
        compiler_params=pltpu.CompilerParams(use_tc_tiling_on_sc=False),
    )(functools.partial(_hop_body, final))


_hop_mid = _make_hop(False)
_hop_fin = _make_hop(True)

_R = 1000


def _mm_body(x_ref, w_ref, d_ref, o_ref, inv_ref, nrm_ref):
    deg = jnp.maximum(d_ref[0] + d_ref[1], 1.0)
    nrm = lax.rsqrt(deg)
    res = jnp.dot(x_ref[...], w_ref[...],
                  preferred_element_type=jnp.float32) * nrm
    o_ref[0] = res[:, :HD]
    o_ref[1] = res[:, HD:]
    inv_ref[...] = jnp.broadcast_to(1.0 / deg, (_R, 16))
    nrm_ref[...] = jnp.broadcast_to(nrm, (_R, 16))


def _scaled_mm(feat, W, d3):
    return pl.pallas_call(
        _mm_body,
        grid=(N // _R,),
        in_specs=[
            pl.BlockSpec((_R, D), lambda i: (i, 0)),
            pl.BlockSpec((D, D), lambda i: (0, 0)),
            pl.BlockSpec((NC, _R, 1), lambda i: (0, i, 0)),
        ],
        out_specs=[
            pl.BlockSpec((NC, _R, HD), lambda i: (0, i, 0)),
            pl.BlockSpec((_R, 16), lambda i: (i, 0)),
            pl.BlockSpec((_R, 16), lambda i: (i, 0)),
        ],
        out_shape=[
            jax.ShapeDtypeStruct((NC, N, HD), jnp.float32),
            jax.ShapeDtypeStruct((N, 16), jnp.float32),
            jax.ShapeDtypeStruct((N, 16), jnp.float32),
        ],
    )(feat, W, d3)


def kernel(feat, edge_index, W, b):
    src0 = edge_index[0].reshape(NS, HCH, HC)
    src = jnp.concatenate([src0, src0 + N])
    dst_h = edge_index[1].reshape(NS, HCH, HC)
    dst_d = edge_index[1].reshape(NW, DCH, C)
    dpart = _deg_kernel(dst_d)
    d3 = dpart.reshape(NC, NPAD, 1)
    g0, inv, nrm = _scaled_mm(feat, W, d3)
    zeros2 = jnp.zeros((NC, HD), jnp.float32)
    p = _hop_mid(g0.reshape(NC * N, HD), src, dst_h, inv, zeros2)
    return _hop_fin(p, src, dst_h, nrm, b.reshape(NC, HD))


# speedup vs baseline: 1.9608x; 1.0264x over previous
import functools

import jax
import jax.numpy as jnp
from jax import lax
from jax.experimental import pallas as pl
from jax.experimental.pallas import tpu as pltpu
from jax.experimental.pallas import tpu_sc as plsc

N = 10000
E = 320000
D = 128
HD = D // 2

NC = 2
NS = 16
NW = NC * NS
C = 80
DCH = E // NW // C
HC = 80
HCH = E // NS // HC
NPAD = 10240
RPS = NPAD // NS
RPS2 = N // NS
DRC = 125
NDR = RPS2 // DRC

_MESH = dict(core_axis_name="c", subcore_axis_name="s", num_cores=NC,
             num_subcores=NS)


@functools.partial(
    pl.kernel,
    out_type=jax.ShapeDtypeStruct((NC * NPAD,), jnp.float32),
    mesh=plsc.VectorSubcoreMesh(**_MESH),
    scratch_types=[
        pltpu.VMEM((DCH, C), jnp.int32),
        pltpu.VMEM((C,), jnp.float32),
        pltpu.VMEM((RPS,), jnp.float32),
        pltpu.VMEM_SHARED((NPAD,), jnp.float32),
    ],
)
def _deg_kernel(dst_hbm, out_hbm, dst_v, ones_v, stage_v, acc):
    c = lax.axis_index("c")
    s = lax.axis_index("s")
    wid = s * NC + c

    one16 = jnp.full((16,), 1.0, jnp.float32)
    zero16 = jnp.zeros((16,), jnp.float32)
    for t in range(C // 16):
        ones_v[pl.ds(t * 16, 16)] = one16

    def zfill(i, carry):
        stage_v[pl.ds(i * 16, 16)] = zero16
        return carry

    lax.fori_loop(0, RPS // 16, zfill, 0)
    pltpu.sync_copy(stage_v, acc.at[pl.ds(s * RPS, RPS)])
    pltpu.sync_copy(dst_hbm.at[wid], dst_v)
    plsc.subcore_barrier()

    def body(j, carry):
        pltpu.sync_copy(ones_v, acc.at[dst_v.at[j]], add=True)
        return carry

    lax.fori_loop(0, DCH, body, 0)
    plsc.subcore_barrier()

    pltpu.sync_copy(acc.at[pl.ds(s * RPS, RPS)], stage_v)
    pltpu.sync_copy(stage_v, out_hbm.at[pl.ds(c * NPAD + s * RPS, RPS)])


def _hop_body(final, g_hbm, src_hbm, dst_hbm, scale_hbm, bias_hbm, out_hbm,
              src_v, dst_v, scale_v, scale2_v, bias_v, rows, stage_v,
              stage2_v, acc, gs, ss):
    c = lax.axis_index("c")
    s = lax.axis_index("s")

    zero16 = jnp.zeros((16,), jnp.float32)

    def zrow(r, carry):
        for t in range(HD // 16):
            stage_v[r, pl.ds(t * 16, 16)] = zero16
        return carry

    lax.fori_loop(0, DRC, zrow, 0)
    for k in range(NDR):
        pltpu.async_copy(stage_v, acc.at[pl.ds(s * RPS2 + k * DRC, DRC)],
                         gs[0])
    pltpu.sync_copy(src_hbm.at[c * NS + s], src_v)
    pltpu.sync_copy(dst_hbm.at[s], dst_v)
    pltpu.sync_copy(bias_hbm.at[c], bias_v)
    for k in range(NDR):
        pltpu.make_async_copy(
            stage_v, acc.at[pl.ds(s * RPS2, DRC)], gs[0]).wait()
    plsc.subcore_barrier()

    def wait_g(k):
        pltpu.make_async_copy(g_hbm.at[src_v.at[0]], rows[k], gs[k]).wait()

    def wait_s(k):
        pltpu.make_async_copy(rows[k], acc.at[dst_v.at[0]], ss[k]).wait()

    def gather(j, k):
        pltpu.async_copy(g_hbm.at[src_v.at[j]], rows[k], gs[k])

    def scatter(j, k):
        pltpu.async_copy(rows[k], acc.at[dst_v.at[j]], ss[k], add=True)

    gather(0, 0)
    gather(1, 1)
    for j in (0, 1):
        wait_g(j)
        scatter(j, j)
        gather(j + 2, j + 2)

    def body(i, carry):
        for m in range(4):
            j = 2 + 4 * i + m
            k = (2 + m) % 4
            k2 = (k + 2) % 4
            wait_g(k)
            scatter(j, k)
            wait_s(k2)
            gather(jnp.minimum(j + 2, HCH - 1), k2)
        return carry

    lax.fori_loop(0, (HCH - 2) // 4, body, 0)
    for k in (0, 1):
        wait_s(k)
    for k in (2, 3):
        wait_g(k)
    plsc.subcore_barrier()

    stages = (stage_v, stage2_v)
    scales = (scale_v, scale2_v)

    def base(k):
        return s * RPS2 + k * DRC

    def out_slice(k):
        if final:
            return out_hbm.at[pl.ds(base(k), DRC), pl.ds(c * HD, HD)]
        return out_hbm.at[pl.ds(c * N + base(k), DRC)]

    def din(k, p):
        pltpu.async_copy(acc.at[pl.ds(base(k), DRC)], stages[p], gs[p])
        pltpu.async_copy(scale_hbm.at[pl.ds(base(k), DRC)], scales[p], ss[p])

    def win(p):
        pltpu.make_async_copy(
            acc.at[pl.ds(base(0), DRC)], stages[p], gs[p]).wait()
        pltpu.make_async_copy(
            scale_hbm.at[pl.ds(base(0), DRC)], scales[p], ss[p]).wait()

    def wout(p):
        pltpu.make_async_copy(stages[p], out_slice(0), gs[2 + p]).wait()

    din(0, 0)
    for k in range(NDR):
        p = k % 2
        if k + 1 < NDR:
            if k >= 1:
                wout(1 - p)
            din(k + 1, 1 - p)
        win(p)

        def scrow(i, carry, p=p):
            for u in range(5):
                r = 5 * i + u
                svec = scales[p][r]
                for t in range(HD // 16):
                    sl = pl.ds(t * 16, 16)
                    stages[p][r, sl] = stages[p][r, sl] * svec + bias_v[sl]
            return carry

        lax.fori_loop(0, DRC // 5, scrow, 0)
        pltpu.async_copy(stages[p], out_slice(k), gs[2 + p])
    wout(0)
    wout(1)


def _make_hop(final):
    return functools.partial(
        pl.kernel,
        out_type=jax.ShapeDtypeStruct(
            (N, D) if final else (NC * N, HD), jnp.float32),
        mesh=plsc.VectorSubcoreMesh(**_MESH),
        scratch_types=[
            pltpu.VMEM((HCH, HC), jnp.int32),
            pltpu.VMEM((HCH, HC), jnp.int32),
            pltpu.VMEM((DRC, 16), jnp.float32),
            pltpu.VMEM((DRC, 16), jnp.float32),
            pltpu.VMEM((HD,), jnp.float32),
            [pltpu.VMEM((HC, HD), jnp.float32) for _ in range(4)],
            pltpu.VMEM((DRC, HD), jnp.float32),
            pltpu.VMEM((DRC, HD), jnp.float32),
            pltpu.VMEM_SHARED((N, HD), jnp.float32),
            [pltpu.SemaphoreType.DMA for _ in range(4)],
            [pltpu.SemaphoreType.DMA for _ in range(4)],
        ],
        compiler_params=pltpu.CompilerParams(use_tc_tiling_on_sc=False),
    )(functools.partial(_hop_body, final))


_hop_mid = _make_hop(False)
_hop_fin = _make_hop(True)

_R = 1000


def _mm_body(x_ref, w_ref, d_ref, o_ref, inv_ref, nrm_ref):
    deg = jnp.maximum(d_ref[0] + d_ref[1], 1.0)
    nrm = lax.rsqrt(deg)
    res = jnp.dot(x_ref[...], w_ref[...],
                  preferred_element_type=jnp.float32) * nrm
    o_ref[0] = res[:, :HD]
    o_ref[1] = res[:, HD:]
    inv_ref[...] = jnp.broadcast_to(1.0 / deg, (_R, 16))
    nrm_ref[...] = jnp.broadcast_to(nrm, (_R, 16))


def _scaled_mm(feat, W, d3):
    return pl.pallas_call(
        _mm_body,
        grid=(N // _R,),
        in_specs=[
            pl.BlockSpec((_R, D), lambda i: (i, 0)),
            pl.BlockSpec((D, D), lambda i: (0, 0)),
            pl.BlockSpec((NC, _R, 1), lambda i: (0, i, 0)),
        ],
        out_specs=[
            pl.BlockSpec((NC, _R, HD), lambda i: (0, i, 0)),
            pl.BlockSpec((_R, 16), lambda i: (i, 0)),
            pl.BlockSpec((_R, 16), lambda i: (i, 0)),
        ],
        out_shape=[
            jax.ShapeDtypeStruct((NC, N, HD), jnp.float32),
            jax.ShapeDtypeStruct((N, 16), jnp.float32),
            jax.ShapeDtypeStruct((N, 16), jnp.float32),
        ],
    )(feat, W, d3)


def kernel(feat, edge_index, W, b):
    src0 = edge_index[0].reshape(NS, HCH, HC)
    src = jnp.concatenate([src0, src0 + N])
    dst_h = edge_index[1].reshape(NS, HCH, HC)
    dst_d = edge_index[1].reshape(NW, DCH, C)
    dpart = _deg_kernel(dst_d)
    d3 = dpart.reshape(NC, NPAD, 1)
    g0, inv, nrm = _scaled_mm(feat, W, d3)
    zeros2 = jnp.zeros((NC, HD), jnp.float32)
    p = _hop_mid(g0.reshape(NC * N, HD), src, dst_h, inv, zeros2)
    return _hop_fin(p, src, dst_h, nrm, b.reshape(NC, HD))


# speedup vs baseline: 2.1946x; 1.1193x over previous
import functools

import jax
import jax.numpy as jnp
from jax import lax
from jax.experimental import pallas as pl
from jax.experimental.pallas import tpu as pltpu
from jax.experimental.pallas import tpu_sc as plsc

N = 10000
E = 320000
D = 128
HD = D // 2

NC = 2
NS = 16
NW = NC * NS
C = 80
DCH = E // NW // C
HC = 80
HCH = E // NS // HC
NPAD = 10240
RPS = NPAD // NS
RPS2 = N // NS
DRC = 125
NDR = RPS2 // DRC

_MESH = dict(core_axis_name="c", subcore_axis_name="s", num_cores=NC,
             num_subcores=NS)


@functools.partial(
    pl.kernel,
    out_type=jax.ShapeDtypeStruct((NC * NPAD,), jnp.float32),
    mesh=plsc.VectorSubcoreMesh(**_MESH),
    scratch_types=[
        pltpu.VMEM((DCH, C), jnp.int32),
        pltpu.VMEM((C,), jnp.float32),
        pltpu.VMEM((RPS,), jnp.float32),
        pltpu.VMEM_SHARED((NPAD,), jnp.float32),
    ],
)
def _deg_kernel(dst_hbm, out_hbm, dst_v, ones_v, stage_v, acc):
    c = lax.axis_index("c")
    s = lax.axis_index("s")
    wid = s * NC + c

    one16 = jnp.full((16,), 1.0, jnp.float32)
    zero16 = jnp.zeros((16,), jnp.float32)
    for t in range(C // 16):
        ones_v[pl.ds(t * 16, 16)] = one16

    def zfill(i, carry):
        stage_v[pl.ds(i * 16, 16)] = zero16
        return carry

    lax.fori_loop(0, RPS // 16, zfill, 0)
    pltpu.sync_copy(stage_v, acc.at[pl.ds(s * RPS, RPS)])
    pltpu.sync_copy(dst_hbm.at[wid], dst_v)
    plsc.subcore_barrier()

    def body(j, carry):
        pltpu.sync_copy(ones_v, acc.at[dst_v.at[j]], add=True)
        return carry

    lax.fori_loop(0, DCH, body, 0)
    plsc.subcore_barrier()

    pltpu.sync_copy(acc.at[pl.ds(s * RPS, RPS)], stage_v)
    pltpu.sync_copy(stage_v, out_hbm.at[pl.ds(c * NPAD + s * RPS, RPS)])


def _hop_body(final, g_hbm, src_hbm, dst_hbm, scale_hbm, bias_hbm, out_hbm,
              src_v, dst_v, scale_v, scale2_v, bias_v, rows, stage_v,
              stage2_v, acc, gs, ss):
    c = lax.axis_index("c")
    s = lax.axis_index("s")

    zero16 = jnp.zeros((16,), jnp.float32)

    def zrow(r, carry):
        for t in range(HD // 16):
            stage_v[r, pl.ds(t * 16, 16)] = zero16
        return carry

    lax.fori_loop(0, DRC, zrow, 0)
    for k in range(NDR):
        pltpu.async_copy(stage_v, acc.at[pl.ds(s * RPS2 + k * DRC, DRC)],
                         gs[0])
    pltpu.sync_copy(src_hbm.at[c * NS + s], src_v)
    pltpu.sync_copy(dst_hbm.at[s], dst_v)
    pltpu.sync_copy(bias_hbm.at[c], bias_v)
    for k in range(NDR):
        pltpu.make_async_copy(
            stage_v, acc.at[pl.ds(s * RPS2, DRC)], gs[0]).wait()
    plsc.subcore_barrier()

    def wait_g(k):
        pltpu.make_async_copy(g_hbm.at[src_v.at[0]], rows[k], gs[k]).wait()

    def wait_s(k):
        pltpu.make_async_copy(rows[k], acc.at[dst_v.at[0]], ss[k]).wait()

    def gather(j, k):
        pltpu.async_copy(g_hbm.at[src_v.at[j]], rows[k], gs[k])

    def scatter(j, k):
        pltpu.async_copy(rows[k], acc.at[dst_v.at[j]], ss[k], add=True)

    for j in (0, 1, 2):
        gather(j, j)
    for j in (0, 1, 2, 3):
        wait_g(j)
        scatter(j, j)
        if j == 3:
            wait_s(0)
        gather(j + 3, (j + 3) % 6)

    def body(i, carry):
        for m in range(6):
            j = 4 + 6 * i + m
            k = (4 + m) % 6
            k3 = (k + 3) % 6
            wait_g(k)
            scatter(j, k)
            wait_s(k3)
            gather(jnp.minimum(j + 3, HCH - 1), k3)
        return carry

    lax.fori_loop(0, (HCH - 4) // 6, body, 0)
    for k in (1, 2, 3):
        wait_s(k)
    for k in (4, 5, 0):
        wait_g(k)
    plsc.subcore_barrier()

    stages = (stage_v, stage2_v)
    scales = (scale_v, scale2_v)

    def base(k):
        return s * RPS2 + k * DRC

    def out_slice(k):
        if final:
            return out_hbm.at[pl.ds(base(k), DRC), pl.ds(c * HD, HD)]
        return out_hbm.at[pl.ds(c * N + base(k), DRC)]

    def din(k, p):
        pltpu.async_copy(acc.at[pl.ds(base(k), DRC)], stages[p], gs[p])
        pltpu.async_copy(scale_hbm.at[pl.ds(base(k), DRC)], scales[p], ss[p])

    def win(p):
        pltpu.make_async_copy(
            acc.at[pl.ds(base(0), DRC)], stages[p], gs[p]).wait()
        pltpu.make_async_copy(
            scale_hbm.at[pl.ds(base(0), DRC)], scales[p], ss[p]).wait()

    def wout(p):
        pltpu.make_async_copy(stages[p], out_slice(0), gs[2 + p]).wait()

    din(0, 0)
    for k in range(NDR):
        p = k % 2
        if k + 1 < NDR:
            if k >= 1:
                wout(1 - p)
            din(k + 1, 1 - p)
        win(p)

        def scrow(i, carry, p=p):
            for u in range(5):
                r = 5 * i + u
                svec = scales[p][r]
                for t in range(HD // 16):
                    sl = pl.ds(t * 16, 16)
                    stages[p][r, sl] = stages[p][r, sl] * svec + bias_v[sl]
            return carry

        lax.fori_loop(0, DRC // 5, scrow, 0)
        pltpu.async_copy(stages[p], out_slice(k), gs[2 + p])
    wout(0)
    wout(1)


def _make_hop(final):
    return functools.partial(
        pl.kernel,
        out_type=jax.ShapeDtypeStruct(
            (N, D) if final else (NC * N, HD), jnp.float32),
        mesh=plsc.VectorSubcoreMesh(**_MESH),
        scratch_types=[
            pltpu.VMEM((HCH, HC), jnp.int32),
            pltpu.VMEM((HCH, HC), jnp.int32),
            pltpu.VMEM((DRC, 16), jnp.float32),
            pltpu.VMEM((DRC, 16), jnp.float32),
            pltpu.VMEM((HD,), jnp.float32),
            [pltpu.VMEM((HC, HD), jnp.float32) for _ in range(6)],
            pltpu.VMEM((DRC, HD), jnp.float32),
            pltpu.VMEM((DRC, HD), jnp.float32),
            pltpu.VMEM_SHARED((N, HD), jnp.float32),
            [pltpu.SemaphoreType.DMA for _ in range(6)],
            [pltpu.SemaphoreType.DMA for _ in range(6)],
        ],
        compiler_params=pltpu.CompilerParams(use_tc_tiling_on_sc=False),
    )(functools.partial(_hop_body, final))


_hop_mid = _make_hop(False)
_hop_fin = _make_hop(True)

_R = 1000


def _mm_body(x_ref, w_ref, d_ref, o_ref, inv_ref, nrm_ref):
    deg = jnp.maximum(d_ref[0] + d_ref[1], 1.0)
    nrm = lax.rsqrt(deg)
    res = jnp.dot(x_ref[...], w_ref[...],
                  preferred_element_type=jnp.float32) * nrm
    o_ref[0] = res[:, :HD]
    o_ref[1] = res[:, HD:]
    inv_ref[...] = jnp.broadcast_to(1.0 / deg, (_R, 16))
    nrm_ref[...] = jnp.broadcast_to(nrm, (_R, 16))


def _scaled_mm(feat, W, d3):
    return pl.pallas_call(
        _mm_body,
        grid=(N // _R,),
        in_specs=[
            pl.BlockSpec((_R, D), lambda i: (i, 0)),
            pl.BlockSpec((D, D), lambda i: (0, 0)),
            pl.BlockSpec((NC, _R, 1), lambda i: (0, i, 0)),
        ],
        out_specs=[
            pl.BlockSpec((NC, _R, HD), lambda i: (0, i, 0)),
            pl.BlockSpec((_R, 16), lambda i: (i, 0)),
            pl.BlockSpec((_R, 16), lambda i: (i, 0)),
        ],
        out_shape=[
            jax.ShapeDtypeStruct((NC, N, HD), jnp.float32),
            jax.ShapeDtypeStruct((N, 16), jnp.float32),
            jax.ShapeDtypeStruct((N, 16), jnp.float32),
        ],
    )(feat, W, d3)


def kernel(feat, edge_index, W, b):
    src0 = edge_index[0].reshape(NS, HCH, HC)
    src = jnp.concatenate([src0, src0 + N])
    dst_h = edge_index[1].reshape(NS, HCH, HC)
    dst_d = edge_index[1].reshape(NW, DCH, C)
    dpart = _deg_kernel(dst_d)
    d3 = dpart.reshape(NC, NPAD, 1)
    g0, inv, nrm = _scaled_mm(feat, W, d3)
    zeros2 = jnp.zeros((NC, HD), jnp.float32)
    p = _hop_mid(g0.reshape(NC * N, HD), src, dst_h, inv, zeros2)
    return _hop_fin(p, src, dst_h, nrm, b.reshape(NC, HD))
